# baseline (speedup 1.0000x reference)
import functools
import math

import jax
import jax.numpy as jnp
from jax import lax
from jax.experimental import pallas as pl
from jax.experimental.pallas import tpu as pltpu
from jax.experimental.pallas import tpu_sc as plsc

D_MODEL = 2048
SCALE = math.sqrt(float(D_MODEL))

_INFO = plsc.get_sparse_core_info()
_NC = _INFO.num_cores
_NS = _INFO.num_subcores
_LANES = _INFO.num_lanes
_NW = _NC * _NS

_CHUNK = 8
_VECS_PER_ROW = D_MODEL // _LANES


def _embed_sc(num_tokens: int):
    rows_per_w = num_tokens // _NW
    n_chunks = rows_per_w // _CHUNK
    mesh = plsc.VectorSubcoreMesh(core_axis_name="c", subcore_axis_name="s")

    @functools.partial(
        pl.kernel,
        mesh=mesh,
        out_type=jax.ShapeDtypeStruct((num_tokens, D_MODEL), jnp.float32),
        scratch_types=[
            pltpu.VMEM((rows_per_w,), jnp.int32),
            pltpu.VMEM((_CHUNK, D_MODEL), jnp.float32),
            pltpu.VMEM((_CHUNK, D_MODEL), jnp.float32),
            pltpu.VMEM((_CHUNK, D_MODEL), jnp.float32),
            pltpu.VMEM((_CHUNK, D_MODEL), jnp.float32),
            pltpu.VMEM((_CHUNK, D_MODEL), jnp.float32),
            pltpu.VMEM((_CHUNK, D_MODEL), jnp.float32),
            pltpu.SemaphoreType.DMA,
            pltpu.SemaphoreType.DMA,
            pltpu.SemaphoreType.DMA,
            pltpu.SemaphoreType.DMA,
            pltpu.SemaphoreType.DMA,
            pltpu.SemaphoreType.DMA,
        ],
    )
    def k(x_hbm, table_hbm, out_hbm, idx_v, g0, g1, g2, g3, s0, s1,
          gsem0, gsem1, gsem2, gsem3, ssem0, ssem1):
        wid = lax.axis_index("s") * _NC + lax.axis_index("c")
        base = wid * rows_per_w
        gbufs = (g0, g1, g2, g3)
        sbufs = (s0, s1)
        gsems = (gsem0, gsem1, gsem2, gsem3)
        ssems = (ssem0, ssem1)

        pltpu.sync_copy(x_hbm.at[pl.ds(base, rows_per_w)], idx_v)

        def start_gather(chunk, slot):
            pltpu.async_copy(
                table_hbm.at[idx_v.at[pl.ds(chunk * _CHUNK, _CHUNK)]],
                gbufs[slot], gsems[slot])

        def wait_gather(slot):
            pltpu.make_async_copy(
                table_hbm.at[pl.ds(0, _CHUNK)], gbufs[slot],
                gsems[slot]).wait()

        def start_scatter(chunk, slot):
            pltpu.async_copy(
                sbufs[slot],
                out_hbm.at[pl.ds(base + chunk * _CHUNK, _CHUNK)],
                ssems[slot])

        def wait_scatter(slot):
            pltpu.make_async_copy(
                out_hbm.at[pl.ds(0, _CHUNK)], sbufs[slot],
                ssems[slot]).wait()

        for j in range(4):
            start_gather(j, j)

        def step(i, carry):
            for t in range(4):
                chunk = i + t
                gslot = t
                sslot = t % 2
                wait_gather(gslot)

                @pl.when(chunk >= 2)
                def _():
                    wait_scatter(sslot)

                gb = gbufs[gslot]
                sb = sbufs[sslot]

                def scale_row(r, c2):
                    for v in range(_VECS_PER_ROW):
                        sl = pl.ds(v * _LANES, _LANES)
                        sb[r, sl] = gb[r, sl] * SCALE
                    return c2

                lax.fori_loop(0, _CHUNK, scale_row, 0)

                @pl.when(chunk + 4 < n_chunks)
                def _():
                    start_gather(chunk + 4, gslot)

                start_scatter(chunk, sslot)
            return carry

        lax.fori_loop(0, n_chunks // 4, lambda j, c: step(j * 4, c), 0)

        wait_scatter(0)
        wait_scatter(1)

    return k


@jax.jit
def kernel(x, table):
    b, s = x.shape
    xf = x.reshape(-1).astype(jnp.int32)
    out = _embed_sc(b * s)(xf, table)
    return out.reshape(b, s, D_MODEL)


# speedup vs baseline: 1.0067x; 1.0067x over previous
import functools
import math

import jax
import jax.numpy as jnp
from jax import lax
from jax.experimental import pallas as pl
from jax.experimental.pallas import tpu as pltpu
from jax.experimental.pallas import tpu_sc as plsc

D_MODEL = 2048
SCALE = math.sqrt(float(D_MODEL))

_INFO = plsc.get_sparse_core_info()
_NC = _INFO.num_cores
_NS = _INFO.num_subcores
_LANES = _INFO.num_lanes
_NW = _NC * _NS

_CHUNK = 8
_HALF = _CHUNK // 2
_NG = 4
_VECS_PER_ROW = D_MODEL // _LANES


def _embed_sc(num_tokens: int):
    rows_per_w = num_tokens // _NW
    n_chunks = rows_per_w // _CHUNK
    mesh = plsc.VectorSubcoreMesh(core_axis_name="c", subcore_axis_name="s")

    @functools.partial(
        pl.kernel,
        mesh=mesh,
        out_type=jax.ShapeDtypeStruct((num_tokens, D_MODEL), jnp.float32),
        scratch_types=[
            pltpu.VMEM((rows_per_w,), jnp.int32),
            pltpu.VMEM((_CHUNK, D_MODEL), jnp.float32),
            pltpu.VMEM((_CHUNK, D_MODEL), jnp.float32),
            pltpu.VMEM((_CHUNK, D_MODEL), jnp.float32),
            pltpu.VMEM((_CHUNK, D_MODEL), jnp.float32),
            pltpu.VMEM_SHARED((_NS, 2, _HALF, D_MODEL), jnp.float32),
            pltpu.SemaphoreType.DMA,
            pltpu.SemaphoreType.DMA,
            pltpu.SemaphoreType.DMA,
            pltpu.SemaphoreType.DMA,
            pltpu.SemaphoreType.DMA,
            pltpu.SemaphoreType.DMA,
            pltpu.SemaphoreType.DMA,
        ],
    )
    def k(x_hbm, table_hbm, out_hbm, idx_v, g0, g1, g2, g3, spm,
          gsem0, gsem1, gsem2, gsem3, xsem, dsem0, dsem1):
        wid = lax.axis_index("s") * _NC + lax.axis_index("c")
        sid = lax.axis_index("s")
        base = wid * rows_per_w
        gbufs = (g0, g1, g2, g3)
        gsems = (gsem0, gsem1, gsem2, gsem3)
        dsems = (dsem0, dsem1)

        pltpu.sync_copy(x_hbm.at[pl.ds(base, rows_per_w)], idx_v)

        def start_gather(chunk, slot):
            pltpu.async_copy(
                table_hbm.at[idx_v.at[pl.ds(chunk * _CHUNK, _CHUNK)]],
                gbufs[slot], gsems[slot])

        def wait_gather(slot):
            pltpu.make_async_copy(
                table_hbm.at[pl.ds(0, _CHUNK)], gbufs[slot],
                gsems[slot]).wait()

        def wait_xbar(slot):
            pltpu.make_async_copy(
                out_hbm.at[pl.ds(0, _HALF)], spm.at[sid, 0], xsem).wait()

        def wait_outdma(h):
            pltpu.make_async_copy(
                out_hbm.at[pl.ds(0, _HALF)], spm.at[sid, h], dsems[h]).wait()

        for j in range(_NG):
            start_gather(j, j)

        def step(i, carry):
            for t in range(_NG):
                chunk = i + t
                gslot = t
                gb = gbufs[gslot]
                wait_gather(gslot)

                def scale_row(r, c2):
                    for v in range(_VECS_PER_ROW):
                        sl = pl.ds(v * _LANES, _LANES)
                        gb[r, sl] = gb[r, sl] * SCALE
                    return c2

                lax.fori_loop(0, _CHUNK, scale_row, 0)

                for h in range(2):
                    @pl.when(chunk >= 1)
                    def _():
                        wait_outdma(h)
                    pltpu.async_copy(gb.at[pl.ds(h * _HALF, _HALF)],
                                     spm.at[sid, h], xsem)

                for h in range(2):
                    wait_xbar(h)
                    pltpu.async_copy(
                        spm.at[sid, h],
                        out_hbm.at[pl.ds(base + chunk * _CHUNK + h * _HALF,
                                         _HALF)],
                        dsems[h])

                @pl.when(chunk + _NG < n_chunks)
                def _():
                    start_gather(chunk + _NG, gslot)
            return carry

        lax.fori_loop(0, n_chunks // _NG, lambda j, c: step(j * _NG, c), 0)

        wait_outdma(0)
        wait_outdma(1)

    return k


@jax.jit
def kernel(x, table):
    b, s = x.shape
    xf = x.reshape(-1).astype(jnp.int32)
    out = _embed_sc(b * s)(xf, table)
    return out.reshape(b, s, D_MODEL)
